# baseline (speedup 1.0000x reference)
import functools

import jax
import jax.numpy as jnp
from jax import lax
from jax.experimental import pallas as pl
from jax.experimental.pallas import tpu as pltpu
from jax.experimental.pallas import tpu_sc as plsc

G = 128
K = 8
C = G * K


def _gather_kernel(n_idx, v_rows, d_words):
    info = plsc.get_sparse_core_info()
    nc, ns = info.num_cores, info.num_subcores
    nw = nc * ns
    per_w = n_idx // nw
    n_chunks = per_w // C
    assert per_w % C == 0

    mesh = plsc.VectorSubcoreMesh(core_axis_name="c", subcore_axis_name="s")

    @functools.partial(
        pl.kernel,
        mesh=mesh,
        compiler_params=pltpu.CompilerParams(use_tc_tiling_on_sc=False),
        out_type=jax.ShapeDtypeStruct((n_idx, d_words), jnp.int32),
        scratch_types=[
            pltpu.VMEM((K, G), jnp.int32),
            pltpu.VMEM((C, d_words), jnp.int32),
            pltpu.SemaphoreType.DMA,
        ],
    )
    def body(idx_hbm, tab_hbm, out_hbm, idx_v, rows_v, sem):
        wid = lax.axis_index("s") * nc + lax.axis_index("c")

        def chunk(c, carry):
            base = wid * per_w + c * C
            pltpu.sync_copy(idx_hbm.at[pl.ds(wid * (per_w // G) + c * K, K)],
                            idx_v)
            cps = [
                pltpu.async_copy(tab_hbm.at[idx_v.at[j]],
                                 rows_v.at[pl.ds(j * G, G)], sem)
                for j in range(K)
            ]
            for cp in cps:
                cp.wait()
            pltpu.sync_copy(rows_v, out_hbm.at[pl.ds(base, C)])
            return carry

        lax.fori_loop(0, n_chunks, chunk, 0)

    return body


def kernel(vid_ids, emb):
    b, h = vid_ids.shape
    v, d = emb.shape
    dw = d // 2
    n = b * h
    tab = lax.bitcast_convert_type(emb.reshape(v, dw, 2), jnp.int32)
    idx = vid_ids.reshape(n // G, G)
    out = _gather_kernel(n, v, dw)(idx, tab)
    return lax.bitcast_convert_type(out, jnp.float16).reshape(b, h, d)


# speedup vs baseline: 2.6322x; 2.6322x over previous
import functools

import jax
import jax.numpy as jnp
from jax import lax
from jax.experimental import pallas as pl
from jax.experimental.pallas import tpu as pltpu
from jax.experimental.pallas import tpu_sc as plsc

BR = 32


def _gather_kernel(b, h, v, d):
    info = plsc.get_sparse_core_info()
    nc, ns = info.num_cores, info.num_subcores
    nw = nc * ns
    bpw = b // nw
    n_chunks = bpw // BR
    assert bpw % BR == 0

    mesh = plsc.VectorSubcoreMesh(core_axis_name="c", subcore_axis_name="s")

    @functools.partial(
        pl.kernel,
        mesh=mesh,
        compiler_params=pltpu.CompilerParams(
            use_tc_tiling_on_sc=False, needs_layout_passes=False),
        out_type=jax.ShapeDtypeStruct((b, h, d), jnp.float16),
        scratch_types=[
            pltpu.VMEM((BR, h), jnp.int32),
            pltpu.VMEM((BR, h, d), jnp.float16),
            pltpu.SemaphoreType.DMA,
        ],
    )
    def body(idx_hbm, tab_hbm, out_hbm, idx_v, rows_v, sem):
        wid = lax.axis_index("s") * nc + lax.axis_index("c")

        def chunk(c, carry):
            b0 = wid * bpw + c * BR
            pltpu.sync_copy(idx_hbm.at[pl.ds(b0, BR)], idx_v)
            cps = [
                pltpu.async_copy(tab_hbm.at[idx_v.at[j]], rows_v.at[j], sem)
                for j in range(BR)
            ]
            for cp in cps:
                cp.wait()
            pltpu.sync_copy(rows_v, out_hbm.at[pl.ds(b0, BR)])
            return carry

        lax.fori_loop(0, n_chunks, chunk, 0)

    return body


def kernel(vid_ids, emb):
    b, h = vid_ids.shape
    v, d = emb.shape
    idx = jnp.where(vid_ids >= 0, vid_ids, 0)
    return _gather_kernel(b, h, v, d)(idx, emb)


# speedup vs baseline: 2.6513x; 1.0073x over previous
import functools

import jax
import jax.numpy as jnp
from jax import lax
from jax.experimental import pallas as pl
from jax.experimental.pallas import tpu as pltpu
from jax.experimental.pallas import tpu_sc as plsc

BR = 32


def _gather_kernel(b, h, v, d):
    info = plsc.get_sparse_core_info()
    nc, ns = info.num_cores, info.num_subcores
    nw = nc * ns
    bpw = b // nw
    n_chunks = bpw // BR
    assert bpw % BR == 0

    mesh = plsc.VectorSubcoreMesh(core_axis_name="c", subcore_axis_name="s")

    @functools.partial(
        pl.kernel,
        mesh=mesh,
        compiler_params=pltpu.CompilerParams(
            use_tc_tiling_on_sc=False, needs_layout_passes=False),
        out_type=jax.ShapeDtypeStruct((b, h, d), jnp.float16),
        scratch_types=[
            pltpu.VMEM((2, BR, h), jnp.int32),
            pltpu.VMEM((2, BR, h, d), jnp.float16),
            pltpu.SemaphoreType.DMA,
            pltpu.SemaphoreType.DMA,
        ],
    )
    def body(idx_hbm, tab_hbm, out_hbm, idx_v, rows_v, sem_g, sem_o):
        wid = lax.axis_index("s") * nc + lax.axis_index("c")

        def fire_gathers(c, s):
            b0 = wid * bpw + c * BR
            pltpu.sync_copy(idx_hbm.at[pl.ds(b0, BR)], idx_v.at[s])
            for j in range(BR):
                pltpu.async_copy(tab_hbm.at[idx_v.at[s].at[j]],
                                 rows_v.at[s].at[j], sem_g)

        def drain_gathers(s):
            for j in range(BR):
                pltpu.make_async_copy(tab_hbm.at[pl.ds(0, h)],
                                      rows_v.at[s].at[j], sem_g).wait()

        def write_out(c, s):
            b0 = wid * bpw + c * BR
            pltpu.async_copy(rows_v.at[s], out_hbm.at[pl.ds(b0, BR)], sem_o)

        def drain_out(c, s):
            b0 = wid * bpw + c * BR
            pltpu.make_async_copy(rows_v.at[s], out_hbm.at[pl.ds(b0, BR)],
                                  sem_o).wait()

        fire_gathers(0, 0)

        def chunk(c, carry):
            s = lax.rem(c, 2)
            drain_gathers(s)

            @pl.when(c >= 1)
            def _():
                drain_out(c - 1, 1 - s)

            @pl.when(c + 1 < n_chunks)
            def _():
                fire_gathers(c + 1, 1 - s)
            write_out(c, s)
            return carry

        lax.fori_loop(0, n_chunks, chunk, 0)
        drain_out(n_chunks - 1, lax.rem(n_chunks - 1, 2))

    return body


def kernel(vid_ids, emb):
    b, h = vid_ids.shape
    v, d = emb.shape
    idx = jnp.where(vid_ids >= 0, vid_ids, 0)
    return _gather_kernel(b, h, v, d)(idx, emb)
